# baseline (speedup 1.0000x reference)
import jax
import jax.numpy as jnp
from jax import lax
from jax.experimental import pallas as pl
from jax.experimental.pallas import tpu as pltpu
from jax.experimental.pallas import tpu_sc as plsc

_NC = 2
_NS = 16
_NW = _NC * _NS
_CH = 80
_WIN = 8
_NBUF = 2


def _pad_edges(e):
    gran = _CH * _WIN
    ew = -(-e // _NW)
    ew = -(-ew // gran) * gran
    return ew


def _make_sc_conv(n, d, ew):
    nchunk = ew // _CH
    nwin = nchunk // _WIN
    npad = n + 16
    _NT = 10
    rpt = n // _NT
    assert nchunk * _CH == ew and nwin * _WIN == nchunk
    assert rpt % 8 == 0 and _NT * rpt == n and _WIN % _NBUF == 0

    mesh = plsc.VectorSubcoreMesh(core_axis_name="c", subcore_axis_name="s")

    def body(h_hbm, src_hbm, dst_hbm, z_hbm, out_hbm,
             src_v, dst_v, rows_v, acc_sh, *sems):
        gsem = sems[:_NBUF]
        ssem = sems[_NBUF:]
        c = lax.axis_index("c")
        s = lax.axis_index("s")
        w = s * _NC + c
        base = s * rpt

        @pl.when(s < _NT)
        def _zero_acc():
            pltpu.sync_copy(z_hbm.at[pl.ds(base, rpt)],
                            acc_sh.at[pl.ds(base, rpt)])

        plsc.subcore_barrier()

        def _win(wi, carry):
            pltpu.sync_copy(src_hbm.at[w * nwin + wi], src_v)
            pltpu.sync_copy(dst_hbm.at[w * nwin + wi], dst_v)
            gd = [None] * _NBUF
            sd = [None] * _NBUF
            for b in range(_NBUF):
                gd[b] = pltpu.async_copy(h_hbm.at[src_v.at[b]],
                                         rows_v.at[b], gsem[b])
            for t in range(_WIN // _NBUF):
                for b in range(_NBUF):
                    j = t * _NBUF + b
                    gd[b].wait()
                    sd[b] = pltpu.async_copy(rows_v.at[b],
                                             acc_sh.at[dst_v.at[j]],
                                             ssem[b], add=True)
                for b in range(_NBUF):
                    sd[b].wait()
                    j = (t + 1) * _NBUF + b
                    if j < _WIN:
                        gd[b] = pltpu.async_copy(h_hbm.at[src_v.at[j]],
                                                 rows_v.at[b], gsem[b])
            return carry
        lax.fori_loop(0, nwin, _win, 0)

        plsc.subcore_barrier()

        @pl.when(s < _NT)
        def _readout():
            pltpu.sync_copy(acc_sh.at[pl.ds(base, rpt)],
                            out_hbm.at[pl.ds(c * n + base, rpt)])

    return pl.kernel(
        body, out_type=jax.ShapeDtypeStruct((_NC * n, d), jnp.float32),
        mesh=mesh,
        scratch_types=[
            pltpu.VMEM((_WIN, _CH), jnp.int32),
            pltpu.VMEM((_WIN, _CH), jnp.int32),
            pltpu.VMEM((_NBUF, _CH, d), jnp.float32),
            pltpu.VMEM_SHARED((npad, d), jnp.float32),
        ] + [pltpu.SemaphoreType.DMA] * (2 * _NBUF))


def _make_sc_deg(n, d, ew):
    nchunk = ew // _CH
    nwin = nchunk // _WIN
    npad = n + 16
    _NT = 10
    rpt = n // _NT

    mesh = plsc.VectorSubcoreMesh(core_axis_name="c", subcore_axis_name="s")

    def body(dst_hbm, z_hbm, ones_hbm, out_hbm, dst_v, ones_v, acc_sh, sem):
        c = lax.axis_index("c")
        s = lax.axis_index("s")
        w = s * _NC + c
        base = s * rpt

        @pl.when(s < _NT)
        def _zero_acc():
            pltpu.sync_copy(z_hbm.at[pl.ds(base, rpt)],
                            acc_sh.at[pl.ds(base, rpt)])
        pltpu.sync_copy(ones_hbm, ones_v)

        plsc.subcore_barrier()

        def _win(wi, carry):
            pltpu.sync_copy(dst_hbm.at[w * nwin + wi], dst_v)
            sd = [None] * _WIN
            for j in range(_WIN):
                sd[j] = pltpu.async_copy(ones_v, acc_sh.at[dst_v.at[j]],
                                         sem, add=True)
            for j in range(_WIN):
                sd[j].wait()
            return carry
        lax.fori_loop(0, nwin, _win, 0)

        plsc.subcore_barrier()

        @pl.when(s < _NT)
        def _readout():
            pltpu.sync_copy(acc_sh.at[pl.ds(base, rpt)],
                            out_hbm.at[pl.ds(c * n + base, rpt)])

    return pl.kernel(
        body, out_type=jax.ShapeDtypeStruct((_NC * n, d), jnp.float32),
        mesh=mesh,
        scratch_types=[
            pltpu.VMEM((_WIN, _CH), jnp.int32),
            pltpu.VMEM((_CH, d), jnp.float32),
            pltpu.VMEM_SHARED((npad, d), jnp.float32),
            pltpu.SemaphoreType.DMA,
        ])


def _lin_body(x_ref, w_ref, b_ref, o_ref):
    o_ref[...] = (jnp.dot(x_ref[...], w_ref[...],
                          preferred_element_type=jnp.float32) + b_ref[...])


def _combine_body(h_ref, p_ref, g_ref, o_ref, r_ref):
    deg = g_ref[0, :, 0:1] + g_ref[1, :, 0:1]
    rdeg = 1.0 / jnp.maximum(deg, 1.0)
    mean = (p_ref[0] + p_ref[1]) * rdeg
    o_ref[...] = jnp.maximum(0.5 * (h_ref[...] + mean), 0.0)
    r_ref[...] = jnp.broadcast_to(rdeg, h_ref.shape)


def _combine_mm_body(h_ref, p_ref, r_ref, w_ref, b_ref, o_ref):
    t = 0.5 * (h_ref[...] + (p_ref[0] + p_ref[1]) * r_ref[...])
    o_ref[...] = jnp.maximum(
        jnp.dot(t, w_ref[...], preferred_element_type=jnp.float32)
        + b_ref[...], 0.0)


def kernel(x, edge_index, W1, b1, W2, b2, W3, b3):
    n, d = x.shape
    e = edge_index.shape[1]
    ew = _pad_edges(e)
    npd = ew * _NW - e
    nwin = (ew // _CH) // _WIN
    src0 = edge_index[0].astype(jnp.int32)
    dst0 = edge_index[1].astype(jnp.int32)
    if npd:
        src0 = jnp.concatenate([src0, jnp.zeros((npd,), jnp.int32)])
        dst0 = jnp.concatenate([dst0, jnp.full((npd,), n, jnp.int32)])
    src = src0.reshape(_NW * nwin, _WIN, _CH)
    dst = dst0.reshape(_NW * nwin, _WIN, _CH)
    z_nd = jnp.zeros((n, d), jnp.float32)
    ones_chd = jnp.ones((_CH, d), jnp.float32)

    deg_k = _make_sc_deg(n, d, ew)
    conv = _make_sc_conv(n, d, ew)

    blk = 1000
    grid = (n // blk,)
    f32 = jnp.float32

    def _spec_h(i):
        return (i, 0)

    def _spec_w(i):
        return (0, 0)

    def _spec_p(i):
        return (0, i, 0)

    lin1 = pl.pallas_call(
        _lin_body, grid=grid,
        in_specs=[pl.BlockSpec((blk, d), _spec_h),
                  pl.BlockSpec((d, d), _spec_w),
                  pl.BlockSpec((1, d), _spec_w)],
        out_specs=pl.BlockSpec((blk, d), _spec_h),
        out_shape=jax.ShapeDtypeStruct((n, d), f32))

    combine = pl.pallas_call(
        _combine_body, grid=grid,
        in_specs=[pl.BlockSpec((blk, d), _spec_h),
                  pl.BlockSpec((_NC, blk, d), _spec_p),
                  pl.BlockSpec((_NC, blk, d), _spec_p)],
        out_specs=[pl.BlockSpec((blk, d), _spec_h),
                   pl.BlockSpec((blk, d), _spec_h)],
        out_shape=[jax.ShapeDtypeStruct((n, d), f32),
                   jax.ShapeDtypeStruct((n, d), f32)])

    combine_mm = pl.pallas_call(
        _combine_mm_body, grid=grid,
        in_specs=[pl.BlockSpec((blk, d), _spec_h),
                  pl.BlockSpec((_NC, blk, d), _spec_p),
                  pl.BlockSpec((blk, d), _spec_h),
                  pl.BlockSpec((d, d), _spec_w),
                  pl.BlockSpec((1, d), _spec_w)],
        out_specs=pl.BlockSpec((blk, d), _spec_h),
        out_shape=jax.ShapeDtypeStruct((n, d), f32))

    h0 = lin1(x, W1, b1.reshape(1, d))
    degp = deg_k(dst, z_nd, ones_chd)
    p1 = conv(h0, src, dst, z_nd)
    h1, rdeg = combine(h0, p1.reshape(_NC, n, d), degp.reshape(_NC, n, d))
    p2 = conv(h1, src, dst, z_nd)
    h2 = combine_mm(h1, p2.reshape(_NC, n, d), rdeg, W2, b2.reshape(1, d))
    p3 = conv(h2, src, dst, z_nd)
    h3 = combine_mm(h2, p3.reshape(_NC, n, d), rdeg, W3, b3.reshape(1, d))
    return h3


# speedup vs baseline: 1.1206x; 1.1206x over previous
import jax
import jax.numpy as jnp
from jax import lax
from jax.experimental import pallas as pl
from jax.experimental.pallas import tpu as pltpu
from jax.experimental.pallas import tpu_sc as plsc

_NC = 2
_NS = 16
_NW = _NC * _NS
_CH = 128
_WIN = 16
_SPARE = 16


def _pad_edges(e):
    gran = _CH * _WIN
    ew = -(-e // _NW)
    return -(-ew // gran) * gran


def _make_sc_conv(n, d, ew):
    nchunk = ew // _CH
    npad = n + _SPARE
    _NT = 10
    rpt = n // _NT
    assert nchunk * _CH == ew
    assert rpt % 8 == 0 and _NT * rpt == n

    mesh = plsc.VectorSubcoreMesh(core_axis_name="c", subcore_axis_name="s")

    def body(h_hbm, src_hbm, dst_hbm, z_hbm, out_hbm,
             src_v, dst_v, rows_v, acc_sh):
        c = lax.axis_index("c")
        s = lax.axis_index("s")
        w = s * _NC + c
        base = s * rpt

        @pl.when(s < _NT)
        def _zero_acc():
            pltpu.sync_copy(z_hbm.at[pl.ds(base, rpt)],
                            acc_sh.at[pl.ds(base, rpt)])
        pltpu.sync_copy(src_hbm.at[w], src_v)
        pltpu.sync_copy(dst_hbm.at[w], dst_v)

        plsc.subcore_barrier()

        def _edge(j, carry):
            pltpu.sync_copy(h_hbm.at[src_v.at[j]], rows_v)
            pltpu.sync_copy(rows_v, acc_sh.at[dst_v.at[j]], add=True)
            return carry
        lax.fori_loop(0, nchunk, _edge, 0)

        plsc.subcore_barrier()

        @pl.when(s < _NT)
        def _readout():
            pltpu.sync_copy(acc_sh.at[pl.ds(base, rpt)],
                            out_hbm.at[pl.ds(c * n + base, rpt)])

    return pl.kernel(
        body, out_type=jax.ShapeDtypeStruct((_NC * n, d), jnp.float32),
        mesh=mesh,
        scratch_types=[
            pltpu.VMEM((nchunk, _CH), jnp.int32),
            pltpu.VMEM((nchunk, _CH), jnp.int32),
            pltpu.VMEM((_CH, d), jnp.float32),
            pltpu.VMEM_SHARED((npad, d), jnp.float32),
        ])


def _make_sc_deg(n, d, ew):
    nchunk = ew // _CH
    nwin = nchunk // _WIN
    npad = n + _SPARE
    _NT = 10
    rpt = n // _NT
    assert nwin * _WIN == nchunk

    mesh = plsc.VectorSubcoreMesh(core_axis_name="c", subcore_axis_name="s")

    def body(dst_hbm, z_hbm, ones_hbm, out_hbm, dst_v, ones_v, acc_sh, sem):
        c = lax.axis_index("c")
        s = lax.axis_index("s")
        w = s * _NC + c
        base = s * rpt

        @pl.when(s < _NT)
        def _zero_acc():
            pltpu.sync_copy(z_hbm.at[pl.ds(base, rpt)],
                            acc_sh.at[pl.ds(base, rpt)])
        pltpu.sync_copy(ones_hbm, ones_v)
        pltpu.sync_copy(dst_hbm.at[w], dst_v)

        plsc.subcore_barrier()

        def _win(wi, carry):
            sd = [None] * _WIN
            for j in range(_WIN):
                sd[j] = pltpu.async_copy(
                    ones_v, acc_sh.at[dst_v.at[wi * _WIN + j]], sem, add=True)
            for j in range(_WIN):
                sd[j].wait()
            return carry
        lax.fori_loop(0, nwin, _win, 0)

        plsc.subcore_barrier()

        @pl.when(s < _NT)
        def _readout():
            pltpu.sync_copy(acc_sh.at[pl.ds(base, rpt)],
                            out_hbm.at[pl.ds(c * n + base, rpt)])

    return pl.kernel(
        body, out_type=jax.ShapeDtypeStruct((_NC * n, d), jnp.float32),
        mesh=mesh,
        scratch_types=[
            pltpu.VMEM((nchunk, _CH), jnp.int32),
            pltpu.VMEM((_CH, d), jnp.float32),
            pltpu.VMEM_SHARED((npad, d), jnp.float32),
            pltpu.SemaphoreType.DMA,
        ])


def _lin_body(x_ref, w_ref, b_ref, o_ref):
    o_ref[...] = (jnp.dot(x_ref[...], w_ref[...],
                          preferred_element_type=jnp.float32) + b_ref[...])


def _combine_body(h_ref, p_ref, g_ref, o_ref, r_ref):
    deg = g_ref[0, :, 0:1] + g_ref[1, :, 0:1]
    rdeg = 1.0 / jnp.maximum(deg, 1.0)
    mean = (p_ref[0] + p_ref[1]) * rdeg
    o_ref[...] = jnp.maximum(0.5 * (h_ref[...] + mean), 0.0)
    r_ref[...] = jnp.broadcast_to(rdeg, h_ref.shape)


def _combine_mm_body(h_ref, p_ref, r_ref, w_ref, b_ref, o_ref):
    t = 0.5 * (h_ref[...] + (p_ref[0] + p_ref[1]) * r_ref[...])
    o_ref[...] = jnp.maximum(
        jnp.dot(t, w_ref[...], preferred_element_type=jnp.float32)
        + b_ref[...], 0.0)


def kernel(x, edge_index, W1, b1, W2, b2, W3, b3):
    n, d = x.shape
    e = edge_index.shape[1]
    ew = _pad_edges(e)
    base_ew = e // _NW
    pad = ew - base_ew
    src0 = edge_index[0].astype(jnp.int32).reshape(_NW, base_ew)
    dst0 = edge_index[1].astype(jnp.int32).reshape(_NW, base_ew)
    if pad:
        psrc = jnp.zeros((_NW, pad), jnp.int32)
        pdst = jnp.broadcast_to(n + (jnp.arange(pad, dtype=jnp.int32)
                                     % _SPARE), (_NW, pad))
        src0 = jnp.concatenate([src0, psrc], axis=1)
        dst0 = jnp.concatenate([dst0, pdst], axis=1)
    src = src0.reshape(_NW, ew // _CH, _CH)
    dst = dst0.reshape(_NW, ew // _CH, _CH)
    z_nd = jnp.zeros((n, d), jnp.float32)
    ones_chd = jnp.ones((_CH, d), jnp.float32)

    deg_k = _make_sc_deg(n, d, ew)
    conv = _make_sc_conv(n, d, ew)

    blk = 1000
    grid = (n // blk,)
    f32 = jnp.float32

    def _spec_h(i):
        return (i, 0)

    def _spec_w(i):
        return (0, 0)

    def _spec_p(i):
        return (0, i, 0)

    lin1 = pl.pallas_call(
        _lin_body, grid=grid,
        in_specs=[pl.BlockSpec((blk, d), _spec_h),
                  pl.BlockSpec((d, d), _spec_w),
                  pl.BlockSpec((1, d), _spec_w)],
        out_specs=pl.BlockSpec((blk, d), _spec_h),
        out_shape=jax.ShapeDtypeStruct((n, d), f32))

    combine = pl.pallas_call(
        _combine_body, grid=grid,
        in_specs=[pl.BlockSpec((blk, d), _spec_h),
                  pl.BlockSpec((_NC, blk, d), _spec_p),
                  pl.BlockSpec((_NC, blk, d), _spec_p)],
        out_specs=[pl.BlockSpec((blk, d), _spec_h),
                   pl.BlockSpec((blk, d), _spec_h)],
        out_shape=[jax.ShapeDtypeStruct((n, d), f32),
                   jax.ShapeDtypeStruct((n, d), f32)])

    combine_mm = pl.pallas_call(
        _combine_mm_body, grid=grid,
        in_specs=[pl.BlockSpec((blk, d), _spec_h),
                  pl.BlockSpec((_NC, blk, d), _spec_p),
                  pl.BlockSpec((blk, d), _spec_h),
                  pl.BlockSpec((d, d), _spec_w),
                  pl.BlockSpec((1, d), _spec_w)],
        out_specs=pl.BlockSpec((blk, d), _spec_h),
        out_shape=jax.ShapeDtypeStruct((n, d), f32))

    h0 = lin1(x, W1, b1.reshape(1, d))
    degp = deg_k(dst, z_nd, ones_chd)
    p1 = conv(h0, src, dst, z_nd)
    h1, rdeg = combine(h0, p1.reshape(_NC, n, d), degp.reshape(_NC, n, d))
    p2 = conv(h1, src, dst, z_nd)
    h2 = combine_mm(h1, p2.reshape(_NC, n, d), rdeg, W2, b2.reshape(1, d))
    p3 = conv(h2, src, dst, z_nd)
    h3 = combine_mm(h2, p3.reshape(_NC, n, d), rdeg, W3, b3.reshape(1, d))
    return h3


# speedup vs baseline: 2.9110x; 2.5976x over previous
import jax
import jax.numpy as jnp
from jax import lax
from jax.experimental import pallas as pl
from jax.experimental.pallas import tpu as pltpu
from jax.experimental.pallas import tpu_sc as plsc

_NC = 2
_NS = 16
_NW = _NC * _NS
_CH = 128
_WIN = 10
_SPARE = 16


def _pad_edges(e):
    gran = _CH * _WIN
    ew = -(-e // _NW)
    return -(-ew // gran) * gran


def _make_sc_conv(n, d, ew):
    nchunk = ew // _CH
    nwin = nchunk // _WIN
    npad = n + _SPARE
    _NT = 10
    rpt = n // _NT
    assert nchunk * _CH == ew and nwin * _WIN == nchunk
    assert rpt % 8 == 0 and _NT * rpt == n

    mesh = plsc.VectorSubcoreMesh(core_axis_name="c", subcore_axis_name="s")

    def body(h_hbm, src_hbm, dst_hbm, z_hbm, out_hbm,
             src_v, dst_v, rows_v, g0, g1, acc_sh):
        gsem = [g0, g1]
        c = lax.axis_index("c")
        s = lax.axis_index("s")
        w = s * _NC + c
        base = s * rpt

        @pl.when(s < _NT)
        def _zero_acc():
            pltpu.sync_copy(z_hbm.at[pl.ds(base, rpt)],
                            acc_sh.at[pl.ds(base, rpt)])

        plsc.subcore_barrier()

        def _win(wi, carry):
            pltpu.sync_copy(src_hbm.at[w * nwin + wi], src_v)
            pltpu.sync_copy(dst_hbm.at[w * nwin + wi], dst_v)
            pltpu.async_copy(h_hbm.at[src_v.at[0]], rows_v.at[0], gsem[0])
            for j in range(_WIN):
                b = j % 2
                pltpu.make_async_copy(h_hbm.at[src_v.at[j]],
                                      rows_v.at[b], gsem[b]).wait()
                if j + 1 < _WIN:
                    pltpu.async_copy(h_hbm.at[src_v.at[j + 1]],
                                     rows_v.at[1 - b], gsem[1 - b])
                pltpu.sync_copy(rows_v.at[b],
                                acc_sh.at[dst_v.at[j]], add=True)
            return carry
        lax.fori_loop(0, nwin, _win, 0)

        plsc.subcore_barrier()

        @pl.when(s < _NT)
        def _readout():
            pltpu.sync_copy(acc_sh.at[pl.ds(base, rpt)],
                            out_hbm.at[pl.ds(c * n + base, rpt)])

    return pl.kernel(
        body, out_type=jax.ShapeDtypeStruct((_NC * n, d), jnp.float32),
        mesh=mesh,
        scratch_types=[
            pltpu.VMEM((_WIN, _CH), jnp.int32),
            pltpu.VMEM((_WIN, _CH), jnp.int32),
            pltpu.VMEM((2, _CH, d), jnp.float32),
            pltpu.SemaphoreType.DMA,
            pltpu.SemaphoreType.DMA,
            pltpu.VMEM_SHARED((npad, d), jnp.float32),
        ])


def _make_sc_deg(n, d, ew):
    nchunk = ew // _CH
    nwin = nchunk // _WIN
    npad = n + _SPARE
    _NT = 10
    rpt = n // _NT
    assert nwin * _WIN == nchunk

    mesh = plsc.VectorSubcoreMesh(core_axis_name="c", subcore_axis_name="s")

    def body(dst_hbm, z_hbm, ones_hbm, out_hbm, dst_v, ones_v, acc_sh, sem):
        c = lax.axis_index("c")
        s = lax.axis_index("s")
        w = s * _NC + c
        base = s * rpt

        @pl.when(s < _NT)
        def _zero_acc():
            pltpu.sync_copy(z_hbm.at[pl.ds(base, rpt)],
                            acc_sh.at[pl.ds(base, rpt)])
        pltpu.sync_copy(ones_hbm, ones_v)

        plsc.subcore_barrier()

        def _win(wi, carry):
            pltpu.sync_copy(dst_hbm.at[w * nwin + wi], dst_v)
            sd = [None] * _WIN
            for j in range(_WIN):
                sd[j] = pltpu.async_copy(
                    ones_v, acc_sh.at[dst_v.at[j]], sem, add=True)
            for j in range(_WIN):
                sd[j].wait()
            return carry
        lax.fori_loop(0, nwin, _win, 0)

        plsc.subcore_barrier()

        @pl.when(s < _NT)
        def _readout():
            pltpu.sync_copy(acc_sh.at[pl.ds(base, rpt)],
                            out_hbm.at[pl.ds(c * n + base, rpt)])

    return pl.kernel(
        body, out_type=jax.ShapeDtypeStruct((_NC * n, d), jnp.float32),
        mesh=mesh,
        scratch_types=[
            pltpu.VMEM((_WIN, _CH), jnp.int32),
            pltpu.VMEM((_CH, d), jnp.float32),
            pltpu.VMEM_SHARED((npad, d), jnp.float32),
            pltpu.SemaphoreType.DMA,
        ])


def _lin_body(x_ref, w_ref, b_ref, o_ref):
    o_ref[...] = (jnp.dot(x_ref[...], w_ref[...],
                          preferred_element_type=jnp.float32) + b_ref[...])


def _combine_body(h_ref, p_ref, g_ref, o_ref, r_ref):
    deg = g_ref[0, :, 0:1] + g_ref[1, :, 0:1]
    rdeg = 1.0 / jnp.maximum(deg, 1.0)
    mean = (p_ref[0] + p_ref[1]) * rdeg
    o_ref[...] = jnp.maximum(0.5 * (h_ref[...] + mean), 0.0)
    r_ref[...] = jnp.broadcast_to(rdeg, h_ref.shape)


def _combine_mm_body(h_ref, p_ref, r_ref, w_ref, b_ref, o_ref):
    t = 0.5 * (h_ref[...] + (p_ref[0] + p_ref[1]) * r_ref[...])
    o_ref[...] = jnp.maximum(
        jnp.dot(t, w_ref[...], preferred_element_type=jnp.float32)
        + b_ref[...], 0.0)


def kernel(x, edge_index, W1, b1, W2, b2, W3, b3):
    n, d = x.shape
    e = edge_index.shape[1]
    ew = _pad_edges(e)
    base_ew = e // _NW
    pad = ew - base_ew
    src0 = edge_index[0].astype(jnp.int32).reshape(_NW, base_ew)
    dst0 = edge_index[1].astype(jnp.int32).reshape(_NW, base_ew)
    if pad:
        psrc = jnp.broadcast_to((jnp.arange(pad, dtype=jnp.int32) * 37) % n,
                                (_NW, pad))
        pdst = jnp.broadcast_to(n + (jnp.arange(pad, dtype=jnp.int32)
                                     % _SPARE), (_NW, pad))
        src0 = jnp.concatenate([src0, psrc], axis=1)
        dst0 = jnp.concatenate([dst0, pdst], axis=1)
    nwin = (ew // _CH) // _WIN
    src = src0.reshape(_NW * nwin, _WIN, _CH)
    dst = dst0.reshape(_NW * nwin, _WIN, _CH)
    z_nd = jnp.zeros((n, d), jnp.float32)
    ones_chd = jnp.ones((_CH, d), jnp.float32)

    deg_k = _make_sc_deg(n, d, ew)
    conv = _make_sc_conv(n, d, ew)

    blk = 1000
    grid = (n // blk,)
    f32 = jnp.float32

    def _spec_h(i):
        return (i, 0)

    def _spec_w(i):
        return (0, 0)

    def _spec_p(i):
        return (0, i, 0)

    lin1 = pl.pallas_call(
        _lin_body, grid=grid,
        in_specs=[pl.BlockSpec((blk, d), _spec_h),
                  pl.BlockSpec((d, d), _spec_w),
                  pl.BlockSpec((1, d), _spec_w)],
        out_specs=pl.BlockSpec((blk, d), _spec_h),
        out_shape=jax.ShapeDtypeStruct((n, d), f32))

    combine = pl.pallas_call(
        _combine_body, grid=grid,
        in_specs=[pl.BlockSpec((blk, d), _spec_h),
                  pl.BlockSpec((_NC, blk, d), _spec_p),
                  pl.BlockSpec((_NC, blk, d), _spec_p)],
        out_specs=[pl.BlockSpec((blk, d), _spec_h),
                   pl.BlockSpec((blk, d), _spec_h)],
        out_shape=[jax.ShapeDtypeStruct((n, d), f32),
                   jax.ShapeDtypeStruct((n, d), f32)])

    combine_mm = pl.pallas_call(
        _combine_mm_body, grid=grid,
        in_specs=[pl.BlockSpec((blk, d), _spec_h),
                  pl.BlockSpec((_NC, blk, d), _spec_p),
                  pl.BlockSpec((blk, d), _spec_h),
                  pl.BlockSpec((d, d), _spec_w),
                  pl.BlockSpec((1, d), _spec_w)],
        out_specs=pl.BlockSpec((blk, d), _spec_h),
        out_shape=jax.ShapeDtypeStruct((n, d), f32))

    h0 = lin1(x, W1, b1.reshape(1, d))
    degp = deg_k(dst, z_nd, ones_chd)
    p1 = conv(h0, src, dst, z_nd)
    h1, rdeg = combine(h0, p1.reshape(_NC, n, d), degp.reshape(_NC, n, d))
    p2 = conv(h1, src, dst, z_nd)
    h2 = combine_mm(h1, p2.reshape(_NC, n, d), rdeg, W2, b2.reshape(1, d))
    p3 = conv(h2, src, dst, z_nd)
    h3 = combine_mm(h2, p3.reshape(_NC, n, d), rdeg, W3, b3.reshape(1, d))
    return h3
